# baseline (speedup 1.0000x reference)
import functools

import jax
import jax.numpy as jnp
from jax.experimental import pallas as pl
from jax.experimental.pallas import tpu as pltpu

BETA = 0.25
K = 1024


def _split2(cb):
    hi = cb.astype(jnp.bfloat16)
    mid = (cb - hi.astype(jnp.float32)).astype(jnp.bfloat16)
    return hi, mid


def _bdot(a, b):
    return jax.lax.dot_general(a, b, (((1,), (0,)), ((), ())),
                               preferred_element_type=jnp.float32)


def _vq_stage(r, cb, cn, cb_hi, cb_mid):
    t = r.shape[0]
    rn = jnp.sum(r * r, axis=1, keepdims=True)
    g = jax.lax.dot_general(r, cb, (((1,), (1,)), ((), ())),
                            preferred_element_type=jnp.float32)
    d = rn + cn - 2.0 * g
    m = jnp.min(d, axis=1, keepdims=True)
    iota_k = jax.lax.broadcasted_iota(jnp.int32, (t, K), 1)
    idx = jnp.min(jnp.where(d == m, iota_k, K), axis=1, keepdims=True)
    onehot = (iota_k == idx).astype(jnp.float32).astype(jnp.bfloat16)
    xq = _bdot(onehot, cb_hi) + _bdot(onehot, cb_mid)
    xq_st = r + (xq - r)
    r_new = r - xq_st
    lsum = jnp.sum((xq - r) ** 2)
    return r_new, xq_st, idx, lsum


def _rvq_body(x_ref, s0_ref, s1_ref, m0_ref, m1_ref, c0_ref, c1_ref,
              sem_ref, col_ref, loss_ref,
              is0_ref, is1_ref, im0_ref, im1_ref, ic0_ref, ic1_ref,
              cn_ref, s_hi, s_mid, mc_hi, mc_mid, *, half, nhalves):
    i = pl.program_id(0)

    @pl.when(i == 0)
    def _init():
        loss_ref[...] = jnp.zeros_like(loss_ref)
        cn_ref[0:1, :] = jnp.sum(s0_ref[...] * s0_ref[...], axis=1)[None, :]
        cn_ref[1:2, :] = jnp.sum(s1_ref[...] * s1_ref[...], axis=1)[None, :]
        cn_ref[2:3, :] = jnp.sum(m0_ref[...] * m0_ref[...], axis=1)[None, :]
        cn_ref[3:4, :] = jnp.sum(m1_ref[...] * m1_ref[...], axis=1)[None, :]
        cn_ref[4:5, :] = jnp.sum(c0_ref[...] * c0_ref[...], axis=1)[None, :]
        cn_ref[5:6, :] = jnp.sum(c1_ref[...] * c1_ref[...], axis=1)[None, :]
        for slot, ref_ in ((0, s0_ref), (1, s1_ref)):
            hi, mid = _split2(ref_[...])
            s_hi[slot] = hi
            s_mid[slot] = mid
        for slot, ref_ in ((0, m0_ref), (1, m1_ref), (2, c0_ref), (3, c1_ref)):
            hi, mid = _split2(ref_[...])
            mc_hi[slot] = hi
            mc_mid[slot] = mid

    tb = x_ref.shape[0]
    th = tb // nhalves
    lsums = [0.0] * 6
    for h in range(nhalves):
        sl = pl.ds(h * th, th)
        x = x_ref[sl, :]

        r, xq0, i0, l0 = _vq_stage(x, s0_ref[...], cn_ref[0:1, :],
                                   s_hi[0], s_mid[0])
        r, xq1, i1, l1 = _vq_stage(r, s1_ref[...], cn_ref[1:2, :],
                                   s_hi[1], s_mid[1])
        xq_sh = xq0 + xq1

        rs = r[:, :half]
        rs, xm0, i2, l2 = _vq_stage(rs, m0_ref[...], cn_ref[2:3, :],
                                    mc_hi[0], mc_mid[0])
        rs, xm1, i3, l3 = _vq_stage(rs, m1_ref[...], cn_ref[3:4, :],
                                    mc_hi[1], mc_mid[1])
        sem_ref[sl, :] = (xq_sh[:, :half] + xm0) + xm1

        rc = r[:, half:]
        rc, xc0, i4, l4 = _vq_stage(rc, c0_ref[...], cn_ref[4:5, :],
                                    mc_hi[2], mc_mid[2])
        rc, xc1, i5, l5 = _vq_stage(rc, c1_ref[...], cn_ref[5:6, :],
                                    mc_hi[3], mc_mid[3])
        col_ref[sl, :] = (xq_sh[:, half:] + xc0) + xc1

        is0_ref[sl, :] = i0
        is1_ref[sl, :] = i1
        im0_ref[sl, :] = i2
        im1_ref[sl, :] = i3
        ic0_ref[sl, :] = i4
        ic1_ref[sl, :] = i5

        for k, s in enumerate((l0, l1, l2, l3, l4, l5)):
            lsums[k] = lsums[k] + s

    row = jax.lax.broadcasted_iota(jnp.int32, loss_ref.shape, 0)
    acc = jnp.zeros(loss_ref.shape, jnp.float32)
    for k, s in enumerate(lsums):
        acc = acc + jnp.where(row == k, s, 0.0)
    loss_ref[...] += acc


@functools.partial(jax.jit, static_argnames=())
def kernel(x, codebook_s0, codebook_s1, codebook_m0, codebook_m1,
           codebook_c0, codebook_c1):
    b, d = x.shape
    half = d // 2
    tb = 1024
    nhalves = 2
    grid = b // tb

    cb_spec_full = pl.BlockSpec((K, d), lambda i: (0, 0))
    cb_spec_half = pl.BlockSpec((K, half), lambda i: (0, 0))

    out_shapes = (
        jax.ShapeDtypeStruct((b, half), jnp.float32),
        jax.ShapeDtypeStruct((b, half), jnp.float32),
        jax.ShapeDtypeStruct((8, 128), jnp.float32),
        jax.ShapeDtypeStruct((b, 1), jnp.int32),
        jax.ShapeDtypeStruct((b, 1), jnp.int32),
        jax.ShapeDtypeStruct((b, 1), jnp.int32),
        jax.ShapeDtypeStruct((b, 1), jnp.int32),
        jax.ShapeDtypeStruct((b, 1), jnp.int32),
        jax.ShapeDtypeStruct((b, 1), jnp.int32),
    )
    half_spec = pl.BlockSpec((tb, half), lambda i: (i, 0))
    idx_spec = pl.BlockSpec((tb, 1), lambda i: (i, 0))
    out_specs = (
        half_spec, half_spec,
        pl.BlockSpec((8, 128), lambda i: (0, 0)),
        idx_spec, idx_spec, idx_spec, idx_spec, idx_spec, idx_spec,
    )

    outs = pl.pallas_call(
        functools.partial(_rvq_body, half=half, nhalves=nhalves),
        grid=(grid,),
        in_specs=[
            pl.BlockSpec((tb, d), lambda i: (i, 0)),
            cb_spec_full, cb_spec_full,
            cb_spec_half, cb_spec_half, cb_spec_half, cb_spec_half,
        ],
        out_specs=out_specs,
        out_shape=out_shapes,
        scratch_shapes=[
            pltpu.VMEM((8, K), jnp.float32),
            pltpu.VMEM((2, K, d), jnp.bfloat16),
            pltpu.VMEM((2, K, d), jnp.bfloat16),
            pltpu.VMEM((4, K, half), jnp.bfloat16),
            pltpu.VMEM((4, K, half), jnp.bfloat16),
        ],
        compiler_params=pltpu.CompilerParams(
            dimension_semantics=("arbitrary",),
        ),
    )(x, codebook_s0, codebook_s1, codebook_m0, codebook_m1,
      codebook_c0, codebook_c1)

    sem_xq, col_xq, loss_sums, i0, i1, i2, i3, i4, i5 = outs

    sums = loss_sums[:6, 0]
    denoms = jnp.array([b * d, b * d, b * half, b * half, b * half, b * half],
                       jnp.float32)
    means = sums / denoms
    losses = BETA * means + means
    mean_losses = jnp.mean(losses)

    semantic_indices = jnp.concatenate([i0, i1, i2, i3], axis=1)
    collaborate_indices = jnp.concatenate([i0, i1, i4, i5], axis=1)
    return (sem_xq, col_xq, mean_losses, semantic_indices, collaborate_indices)


# speedup vs baseline: 1.0081x; 1.0081x over previous
import functools

import jax
import jax.numpy as jnp
from jax.experimental import pallas as pl
from jax.experimental.pallas import tpu as pltpu

BETA = 0.25
K = 1024


def _split2(cb):
    hi = cb.astype(jnp.bfloat16)
    mid = (cb - hi.astype(jnp.float32)).astype(jnp.bfloat16)
    return hi, mid


def _bdot(a, b):
    return jax.lax.dot_general(a, b, (((1,), (0,)), ((), ())),
                               preferred_element_type=jnp.float32)


def _vq_stage(r, cb, cn, cb_hi, cb_mid):
    t = r.shape[0]
    rn = jnp.sum(r * r, axis=1, keepdims=True)
    g = jax.lax.dot_general(r, cb, (((1,), (1,)), ((), ())),
                            preferred_element_type=jnp.float32)
    d = rn + cn - 2.0 * g
    m = jnp.min(d, axis=1, keepdims=True)
    iota_k = jax.lax.broadcasted_iota(jnp.int32, (t, K), 1)
    idx = jnp.min(jnp.where(d == m, iota_k, K), axis=1, keepdims=True)
    onehot = (iota_k == idx).astype(jnp.float32).astype(jnp.bfloat16)
    xq = _bdot(onehot, cb_hi) + _bdot(onehot, cb_mid)
    xq_st = r + (xq - r)
    r_new = r - xq_st
    lsum = jnp.sum((xq - r) ** 2)
    return r_new, xq_st, idx, lsum


def _rvq_body(x_ref, s0_ref, s1_ref, m0_ref, m1_ref, c0_ref, c1_ref,
              sem_ref, col_ref, loss_ref,
              is0_ref, is1_ref, im0_ref, im1_ref, ic0_ref, ic1_ref,
              cn_ref, s_hi, s_mid, mc_hi, mc_mid, *, half, nhalves):
    i = pl.program_id(0)

    @pl.when(i == 0)
    def _init():
        loss_ref[...] = jnp.zeros_like(loss_ref)
        cn_ref[0:1, :] = jnp.sum(s0_ref[...] * s0_ref[...], axis=1)[None, :]
        cn_ref[1:2, :] = jnp.sum(s1_ref[...] * s1_ref[...], axis=1)[None, :]
        cn_ref[2:3, :] = jnp.sum(m0_ref[...] * m0_ref[...], axis=1)[None, :]
        cn_ref[3:4, :] = jnp.sum(m1_ref[...] * m1_ref[...], axis=1)[None, :]
        cn_ref[4:5, :] = jnp.sum(c0_ref[...] * c0_ref[...], axis=1)[None, :]
        cn_ref[5:6, :] = jnp.sum(c1_ref[...] * c1_ref[...], axis=1)[None, :]
        for slot, ref_ in ((0, s0_ref), (1, s1_ref)):
            hi, mid = _split2(ref_[...])
            s_hi[slot] = hi
            s_mid[slot] = mid
        for slot, ref_ in ((0, m0_ref), (1, m1_ref), (2, c0_ref), (3, c1_ref)):
            hi, mid = _split2(ref_[...])
            mc_hi[slot] = hi
            mc_mid[slot] = mid

    tb = x_ref.shape[0]
    th = tb // nhalves
    lsums = [0.0] * 6
    for h in range(nhalves):
        sl = pl.ds(h * th, th)
        x = x_ref[sl, :]

        r, xq0, i0, l0 = _vq_stage(x, s0_ref[...], cn_ref[0:1, :],
                                   s_hi[0], s_mid[0])
        r, xq1, i1, l1 = _vq_stage(r, s1_ref[...], cn_ref[1:2, :],
                                   s_hi[1], s_mid[1])
        xq_sh = xq0 + xq1

        rs = r[:, :half]
        rs, xm0, i2, l2 = _vq_stage(rs, m0_ref[...], cn_ref[2:3, :],
                                    mc_hi[0], mc_mid[0])
        rs, xm1, i3, l3 = _vq_stage(rs, m1_ref[...], cn_ref[3:4, :],
                                    mc_hi[1], mc_mid[1])
        sem_ref[sl, :] = (xq_sh[:, :half] + xm0) + xm1

        rc = r[:, half:]
        rc, xc0, i4, l4 = _vq_stage(rc, c0_ref[...], cn_ref[4:5, :],
                                    mc_hi[2], mc_mid[2])
        rc, xc1, i5, l5 = _vq_stage(rc, c1_ref[...], cn_ref[5:6, :],
                                    mc_hi[3], mc_mid[3])
        col_ref[sl, :] = (xq_sh[:, half:] + xc0) + xc1

        is0_ref[sl, :] = i0
        is1_ref[sl, :] = i1
        im0_ref[sl, :] = i2
        im1_ref[sl, :] = i3
        ic0_ref[sl, :] = i4
        ic1_ref[sl, :] = i5

        for k, s in enumerate((l0, l1, l2, l3, l4, l5)):
            lsums[k] = lsums[k] + s

    row = jax.lax.broadcasted_iota(jnp.int32, loss_ref.shape, 0)
    acc = jnp.zeros(loss_ref.shape, jnp.float32)
    for k, s in enumerate(lsums):
        acc = acc + jnp.where(row == k, s, 0.0)
    loss_ref[...] += acc


@functools.partial(jax.jit, static_argnames=())
def kernel(x, codebook_s0, codebook_s1, codebook_m0, codebook_m1,
           codebook_c0, codebook_c1):
    b, d = x.shape
    half = d // 2
    tb = 1024
    nhalves = 1
    grid = b // tb

    cb_spec_full = pl.BlockSpec((K, d), lambda i: (0, 0))
    cb_spec_half = pl.BlockSpec((K, half), lambda i: (0, 0))

    out_shapes = (
        jax.ShapeDtypeStruct((b, half), jnp.float32),
        jax.ShapeDtypeStruct((b, half), jnp.float32),
        jax.ShapeDtypeStruct((8, 128), jnp.float32),
        jax.ShapeDtypeStruct((b, 1), jnp.int32),
        jax.ShapeDtypeStruct((b, 1), jnp.int32),
        jax.ShapeDtypeStruct((b, 1), jnp.int32),
        jax.ShapeDtypeStruct((b, 1), jnp.int32),
        jax.ShapeDtypeStruct((b, 1), jnp.int32),
        jax.ShapeDtypeStruct((b, 1), jnp.int32),
    )
    half_spec = pl.BlockSpec((tb, half), lambda i: (i, 0))
    idx_spec = pl.BlockSpec((tb, 1), lambda i: (i, 0))
    out_specs = (
        half_spec, half_spec,
        pl.BlockSpec((8, 128), lambda i: (0, 0)),
        idx_spec, idx_spec, idx_spec, idx_spec, idx_spec, idx_spec,
    )

    outs = pl.pallas_call(
        functools.partial(_rvq_body, half=half, nhalves=nhalves),
        grid=(grid,),
        in_specs=[
            pl.BlockSpec((tb, d), lambda i: (i, 0)),
            cb_spec_full, cb_spec_full,
            cb_spec_half, cb_spec_half, cb_spec_half, cb_spec_half,
        ],
        out_specs=out_specs,
        out_shape=out_shapes,
        scratch_shapes=[
            pltpu.VMEM((8, K), jnp.float32),
            pltpu.VMEM((2, K, d), jnp.bfloat16),
            pltpu.VMEM((2, K, d), jnp.bfloat16),
            pltpu.VMEM((4, K, half), jnp.bfloat16),
            pltpu.VMEM((4, K, half), jnp.bfloat16),
        ],
        compiler_params=pltpu.CompilerParams(
            dimension_semantics=("arbitrary",),
        ),
    )(x, codebook_s0, codebook_s1, codebook_m0, codebook_m1,
      codebook_c0, codebook_c1)

    sem_xq, col_xq, loss_sums, i0, i1, i2, i3, i4, i5 = outs

    sums = loss_sums[:6, 0]
    denoms = jnp.array([b * d, b * d, b * half, b * half, b * half, b * half],
                       jnp.float32)
    means = sums / denoms
    losses = BETA * means + means
    mean_losses = jnp.mean(losses)

    semantic_indices = jnp.concatenate([i0, i1, i2, i3], axis=1)
    collaborate_indices = jnp.concatenate([i0, i1, i4, i5], axis=1)
    return (sem_xq, col_xq, mean_losses, semantic_indices, collaborate_indices)
